# baseline (speedup 1.0000x reference)
import functools

import jax
import jax.numpy as jnp
from jax import lax
from jax.experimental import pallas as pl
from jax.experimental.pallas import tpu as pltpu
from jax.experimental.pallas import tpu_sc as plsc

N = 10000
E = 320000
NSC = 2
NTEC = 16
EPT = E // NTEC
HN = N // NSC
QN = HN // 2
TRASH = QN
AROWS = 2512
ORPT = AROWS // NTEC
PIECE = 2000
BATCH = 64
NRB = 6
RING = NRB * BATCH
CAPW = 21120
CAPR = CAPW // BATCH
BR = 1000
NB = N // BR


def _sc_segsum(table, src2, dst2, zrows, K):
    mesh = plsc.VectorSubcoreMesh(core_axis_name="c", subcore_axis_name="s")

    @functools.partial(
        pl.kernel,
        out_type=jax.ShapeDtypeStruct((2 * K, N, 128), jnp.float32),
        mesh=mesh,
        scratch_types=[
            pltpu.VMEM((PIECE,), jnp.int32),
            pltpu.VMEM((PIECE,), jnp.int32),
            pltpu.VMEM((CAPR, BATCH), jnp.int32),
            pltpu.VMEM((CAPR, BATCH), jnp.int32),
            [pltpu.VMEM((BATCH, 128), jnp.float32) for _ in range(NRB)],
            pltpu.VMEM_SHARED((AROWS, 128), jnp.float32),
            [pltpu.SemaphoreType.DMA for _ in range(NRB)],
            [pltpu.SemaphoreType.DMA for _ in range(NRB)],
        ],
        compiler_params=pltpu.CompilerParams(
            use_tc_tiling_on_sc=False, needs_layout_passes=False),
    )
    def k(tab_hbm, src_hbm, dst_hbm, z_hbm, out_hbm, srcp_v, dstp_v, srch_v,
          dsth_v, rows, acc_sh, sem_g, sem_s):
        c = lax.axis_index("c")
        s = lax.axis_index("s")
        lo = c * HN

        tr_s = jnp.zeros((16,), jnp.int32)
        tr_d = jnp.full((16,), TRASH, jnp.int32)

        def tfill(r, carry):
            for j in range(BATCH // 16):
                srch_v[r, pl.ds(j * 16, 16)] = tr_s
                dsth_v[r, pl.ds(j * 16, 16)] = tr_d
            return carry

        lax.fori_loop(0, CAPR, tfill, 0)

        def piece(p, cnts):
            pltpu.sync_copy(src_hbm.at[s, pl.ds(p * PIECE, PIECE)], srcp_v)
            pltpu.sync_copy(dst_hbm.at[s, pl.ds(p * PIECE, PIECE)], dstp_v)

            def vec(i, cnts2):
                ca, cb = cnts2
                sv = srcp_v[pl.ds(i * 16, 16)]
                dv = dstp_v[pl.ds(i * 16, 16)]
                dloc = dv - lo
                ma = (dloc >= 0) & (dloc < QN)
                mb = (dloc >= QN) & (dloc < HN)
                mia = jnp.where(ma, 1, 0)
                mib = jnp.where(mb, 1, 0)
                posa = ca + plsc.cumsum(mia) - mia
                posb = CAPW - cb - plsc.cumsum(mib)
                ra, cola = posa >> 6, posa & 63
                rb, colb = posb >> 6, posb & 63
                plsc.store_scatter(srch_v, [ra, cola], sv, mask=ma)
                plsc.store_scatter(dsth_v, [ra, cola], dloc, mask=ma)
                plsc.store_scatter(srch_v, [rb, colb], sv, mask=mb)
                plsc.store_scatter(dsth_v, [rb, colb], dloc - QN, mask=mb)
                return ca + jnp.sum(mia), cb + jnp.sum(mib)

            return lax.fori_loop(0, PIECE // 16, vec, cnts)

        ca, cb = lax.fori_loop(0, EPT // PIECE, piece,
                               (jnp.int32(0), jnp.int32(0)))

        pada = jnp.maximum(((ca + (RING - 1)) // RING) * RING, RING)
        padb = jnp.maximum(((cb + (RING - 1)) // RING) * RING, RING)
        nring_a = pada // RING
        nring_b = padb // RING
        base_b = (CAPW - padb) // BATCH

        for ki in range(2 * K):
            tab_c = tab_hbm.at[ki]
            out_c = out_hbm.at[ki]

            for base, nring, qoff in ((0, nring_a, 0), (base_b, nring_b, QN)):
                nsteps = nring * NRB

                pltpu.sync_copy(z_hbm, acc_sh.at[pl.ds(s * ORPT, ORPT)])
                plsc.subcore_barrier()

                for pb in range(3):
                    pltpu.async_copy(
                        tab_c.at[srch_v.at[base + pb]], rows[pb], sem_g[pb])

                def ring_fn(i0, carry, base=base, nsteps=nsteps, tab_c=tab_c):
                    for b in range(NRB):
                        i = i0 + b
                        bn = (b + 3) % NRB

                        @pl.when(i >= 3)
                        def _():
                            pltpu.make_async_copy(
                                rows[bn], acc_sh.at[dsth_v.at[base + i - 3]],
                                sem_s[bn]).wait()

                        @pl.when(i + 3 < nsteps)
                        def _():
                            pltpu.async_copy(
                                tab_c.at[srch_v.at[base + i + 3]],
                                rows[bn], sem_g[bn])

                        pltpu.make_async_copy(
                            tab_c.at[srch_v.at[base + i]],
                            rows[b], sem_g[b]).wait()
                        pltpu.async_copy(
                            rows[b], acc_sh.at[dsth_v.at[base + i]],
                            sem_s[b], add=True)
                    return carry

                lax.fori_loop(0, nring,
                              lambda j, cr, fn=ring_fn: fn(j * NRB, cr), 0,
                              unroll=False)
                for pb in range(3):
                    pltpu.make_async_copy(
                        rows[3 + pb],
                        acc_sh.at[dsth_v.at[base + nsteps - 3 + pb]],
                        sem_s[3 + pb]).wait()
                plsc.subcore_barrier()

                @pl.when(s < NTEC - 1)
                def _():
                    for off, sz in ((0, 112), (112, 45)):
                        pltpu.sync_copy(
                            acc_sh.at[pl.ds(s * ORPT + off, sz)],
                            out_c.at[pl.ds(lo + qoff + s * ORPT + off, sz)])

                @pl.when(s == NTEC - 1)
                def _():
                    for off, sz in ((0, 112), (112, 33)):
                        pltpu.sync_copy(
                            acc_sh.at[pl.ds(s * ORPT + off, sz)],
                            out_c.at[pl.ds(lo + qoff + s * ORPT + off, sz)])

                if True:
                    plsc.subcore_barrier()

    return k(table, src2, dst2, zrows)


def _tc_colmax(x, t):
    D = x.shape[1]

    def body(x_ref, t_ref, m_ref, mx_ref):
        i = pl.program_id(0)
        pm = jnp.max(x_ref[...] * t_ref[...], axis=0, keepdims=True)

        @pl.when(i == 0)
        def _():
            mx_ref[...] = pm

        @pl.when(i > 0)
        def _():
            mx_ref[...] = jnp.maximum(mx_ref[...], pm)

        m_ref[...] = mx_ref[...]

    return pl.pallas_call(
        body,
        grid=(NB,),
        in_specs=[
            pl.BlockSpec((BR, D), lambda i: (i, 0)),
            pl.BlockSpec((1, D), lambda i: (0, 0)),
        ],
        out_specs=pl.BlockSpec((1, D), lambda i: (0, 0)),
        out_shape=jax.ShapeDtypeStruct((1, D), jnp.float32),
        scratch_shapes=[pltpu.VMEM((1, D), jnp.float32)],
    )(x, t)


def _tc_table(h, t, M, K):
    def body(h_ref, t_ref, m_ref, o_ref):
        k = pl.program_id(1)
        hb = h_ref[...]
        e = jnp.exp(hb * t_ref[...] - m_ref[...])
        o_ref[0] = jnp.where(k < K, e, hb * e)

    return pl.pallas_call(
        body,
        grid=(NB, 2 * K),
        in_specs=[
            pl.BlockSpec((BR, 128), lambda i, k: (i, lax.rem(k, K))),
            pl.BlockSpec((1, 128), lambda i, k: (0, lax.rem(k, K))),
            pl.BlockSpec((1, 128), lambda i, k: (0, lax.rem(k, K))),
        ],
        out_specs=pl.BlockSpec((1, BR, 128), lambda i, k: (k, i, 0)),
        out_shape=jax.ShapeDtypeStruct((2 * K, N, 128), jnp.float32),
    )(h, t, M)


def _tc_sage(SA, x, Wl, b, Wr, K, H):
    D = K * 128

    def body(sa_ref, x_ref, wl_ref, b_ref, wr_ref, z_ref, s_ref, q_ref,
             acc_ref):
        i = pl.program_id(0)
        z = jnp.dot(x_ref[...], wr_ref[...],
                    preferred_element_type=jnp.float32)
        for ki in range(K):
            aggr = sa_ref[K + ki] / (sa_ref[ki] + 1e-16)
            z += jnp.dot(aggr, wl_ref[pl.ds(ki * 128, 128), :],
                         preferred_element_type=jnp.float32)
        z += b_ref[...]
        z_ref[...] = z
        ps = jnp.sum(z)
        pq = jnp.sum(z * z)

        @pl.when(i == 0)
        def _():
            acc_ref[0] = ps
            acc_ref[1] = pq

        @pl.when(i > 0)
        def _():
            acc_ref[0] += ps
            acc_ref[1] += pq

        s_ref[0, 0] = acc_ref[0]
        q_ref[0, 0] = acc_ref[1]

    return pl.pallas_call(
        body,
        grid=(NB,),
        in_specs=[
            pl.BlockSpec((2 * K, BR, 128), lambda i: (0, i, 0)),
            pl.BlockSpec((BR, D), lambda i: (i, 0)),
            pl.BlockSpec((D, H), lambda i: (0, 0)),
            pl.BlockSpec((1, H), lambda i: (0, 0)),
            pl.BlockSpec((D, H), lambda i: (0, 0)),
        ],
        out_specs=[
            pl.BlockSpec((BR, H), lambda i: (i, 0)),
            pl.BlockSpec(memory_space=pltpu.SMEM),
            pl.BlockSpec(memory_space=pltpu.SMEM),
        ],
        out_shape=[
            jax.ShapeDtypeStruct((N, H), jnp.float32),
            jax.ShapeDtypeStruct((1, 1), jnp.float32),
            jax.ShapeDtypeStruct((1, 1), jnp.float32),
        ],
        scratch_shapes=[pltpu.SMEM((2,), jnp.float32)],
    )(SA, x, Wl, b, Wr)


def _graph_ln(z_ref, s_ref, q_ref, w_ref, bb_ref, cnt):
    mean = s_ref[0, 0] / cnt
    var = q_ref[0, 0] / cnt - mean * mean
    std = jnp.sqrt(jnp.maximum(var, 0.0))
    zc = (z_ref[...] - mean) / (std + 1e-5)
    return jnp.maximum(zc * w_ref[...] + bb_ref[...], 0.0)


def _tc_ln_relu_colmax(Z, s, q, ln_w, ln_b, t, H):
    cnt = float(N * H)

    def body(z_ref, s_ref, q_ref, w_ref, bb_ref, t_ref, h_ref, m_ref, mx_ref):
        i = pl.program_id(0)
        h = _graph_ln(z_ref, s_ref, q_ref, w_ref, bb_ref, cnt)
        h_ref[...] = h
        pm = jnp.max(h * t_ref[...], axis=0, keepdims=True)

        @pl.when(i == 0)
        def _():
            mx_ref[...] = pm

        @pl.when(i > 0)
        def _():
            mx_ref[...] = jnp.maximum(mx_ref[...], pm)

        m_ref[...] = mx_ref[...]

    return pl.pallas_call(
        body,
        grid=(NB,),
        in_specs=[
            pl.BlockSpec((BR, H), lambda i: (i, 0)),
            pl.BlockSpec(memory_space=pltpu.SMEM),
            pl.BlockSpec(memory_space=pltpu.SMEM),
            pl.BlockSpec((1, H), lambda i: (0, 0)),
            pl.BlockSpec((1, H), lambda i: (0, 0)),
            pl.BlockSpec((1, H), lambda i: (0, 0)),
        ],
        out_specs=[
            pl.BlockSpec((BR, H), lambda i: (i, 0)),
            pl.BlockSpec((1, H), lambda i: (0, 0)),
        ],
        out_shape=[
            jax.ShapeDtypeStruct((N, H), jnp.float32),
            jax.ShapeDtypeStruct((1, H), jnp.float32),
        ],
        scratch_shapes=[pltpu.VMEM((1, H), jnp.float32)],
    )(Z, s, q, ln_w, ln_b, t)


def _tc_ln_relu_colsum(Z, s, q, ln_w, ln_b, H):
    cnt = float(N * H)

    def body(z_ref, s_ref, q_ref, w_ref, bb_ref, cs_ref, acc_ref):
        i = pl.program_id(0)
        h = _graph_ln(z_ref, s_ref, q_ref, w_ref, bb_ref, cnt)
        pc = jnp.sum(h, axis=0, keepdims=True)

        @pl.when(i == 0)
        def _():
            acc_ref[...] = pc

        @pl.when(i > 0)
        def _():
            acc_ref[...] += pc

        cs_ref[...] = acc_ref[...]

    return pl.pallas_call(
        body,
        grid=(NB,),
        in_specs=[
            pl.BlockSpec((BR, H), lambda i: (i, 0)),
            pl.BlockSpec(memory_space=pltpu.SMEM),
            pl.BlockSpec(memory_space=pltpu.SMEM),
            pl.BlockSpec((1, H), lambda i: (0, 0)),
            pl.BlockSpec((1, H), lambda i: (0, 0)),
        ],
        out_specs=pl.BlockSpec((1, H), lambda i: (0, 0)),
        out_shape=jax.ShapeDtypeStruct((1, H), jnp.float32),
        scratch_shapes=[pltpu.VMEM((1, H), jnp.float32)],
    )(Z, s, q, ln_w, ln_b)


def _tc_head(h2sum, fx_w, fx_b, nx_w, nx_b):
    OUT = fx_w.shape[1]

    def body(cs_ref, w_ref, b_ref, nw_ref, nb_ref, y_ref):
        g = cs_ref[...] / float(N)
        y = jnp.sum(w_ref[...] * g[0][:, None], axis=0,
                    keepdims=True) + b_ref[...]
        mu = jnp.mean(y)
        var = jnp.mean((y - mu) * (y - mu))
        y = (y - mu) / jnp.sqrt(var + 1e-5) * nw_ref[...] + nb_ref[...]
        y_ref[...] = jnp.maximum(y, 0.0)

    return pl.pallas_call(
        body,
        out_shape=jax.ShapeDtypeStruct((1, OUT), jnp.float32),
    )(h2sum, fx_w, fx_b, nx_w, nx_b)


def kernel(x, edge_index, t1, W1l, b1, W1r, ln1_w, ln1_b, t2, W2l, b2, W2r,
           ln2_w, ln2_b, fx_w, fx_b, nx_w, nx_b):
    src2 = edge_index[0].reshape(NTEC, EPT)
    dst2 = edge_index[1].reshape(NTEC, EPT)
    zrows = jnp.zeros((ORPT, 128), jnp.float32)
    r2 = lambda v: v.reshape(1, -1)

    M1 = _tc_colmax(x, t1)
    T1 = _tc_table(x, t1, M1, K=1)
    SA1 = _sc_segsum(T1, src2, dst2, zrows, K=1)
    Z1, s1, q1 = _tc_sage(SA1, x, W1l, r2(b1), W1r, K=1, H=512)
    h1, M2 = _tc_ln_relu_colmax(Z1, s1, q1, r2(ln1_w), r2(ln1_b), t2, H=512)

    T2 = _tc_table(h1, t2, M2, K=4)
    SA2 = _sc_segsum(T2, src2, dst2, zrows, K=4)
    Z2, s2, q2 = _tc_sage(SA2, h1, W2l, r2(b2), W2r, K=4, H=256)
    h2sum = _tc_ln_relu_colsum(Z2, s2, q2, r2(ln2_w), r2(ln2_b), H=256)

    return _tc_head(h2sum, fx_w, r2(fx_b), r2(nx_w), r2(nx_b))


# speedup vs baseline: 4.9770x; 4.9770x over previous
import functools

import jax
import jax.numpy as jnp
from jax import lax
from jax.experimental import pallas as pl
from jax.experimental.pallas import tpu as pltpu
from jax.experimental.pallas import tpu_sc as plsc

N = 10000
E = 320000
CC = 64
NSC = 2
NTEC = 16
EPT = E // NTEC
BATCH = 125
STEPS = EPT // BATCH
RPT = N // NTEC
ZROWS = 125
BR = 1000
NB = N // BR


def _sc_segsum(table, src2, dst2, K):
    mesh = plsc.VectorSubcoreMesh(core_axis_name="c", subcore_axis_name="s")

    @functools.partial(
        pl.kernel,
        out_type=jax.ShapeDtypeStruct((2 * K, N, CC), jnp.float32),
        mesh=mesh,
        scratch_types=[
            pltpu.VMEM((STEPS, BATCH), jnp.int32),
            pltpu.VMEM((STEPS, BATCH), jnp.int32),
            [pltpu.VMEM((BATCH, CC), jnp.float32) for _ in range(4)],
            pltpu.VMEM((ZROWS, CC), jnp.float32),
            pltpu.VMEM_SHARED((N, CC), jnp.float32),
            [pltpu.SemaphoreType.DMA for _ in range(4)],
            [pltpu.SemaphoreType.DMA for _ in range(4)],
        ],
        compiler_params=pltpu.CompilerParams(use_tc_tiling_on_sc=False),
    )
    def k(tab_hbm, src_hbm, dst_hbm, out_hbm, src_v, dst_v, rows, zero_v,
          acc_sh, sem_g, sem_s):
        c = lax.axis_index("c")
        s = lax.axis_index("s")
        pltpu.sync_copy(src_hbm.at[s], src_v)
        pltpu.sync_copy(dst_hbm.at[s], dst_v)

        zeros16 = jnp.zeros((16,), jnp.float32)

        def zrow(r, carry):
            def zcol(cc, carry2):
                zero_v[r, pl.ds(cc * 16, 16)] = zeros16
                return carry2
            return lax.fori_loop(0, CC // 16, zcol, carry)

        lax.fori_loop(0, ZROWS, zrow, 0)

        for ki in range(K):
            chunk = c * K + ki
            tab_c = tab_hbm.at[chunk]

            def zinit(j, carry):
                pltpu.sync_copy(
                    zero_v, acc_sh.at[pl.ds(s * RPT + j * ZROWS, ZROWS)])
                return carry

            lax.fori_loop(0, RPT // ZROWS, zinit, 0)
            plsc.subcore_barrier()

            pltpu.async_copy(tab_c.at[src_v.at[0]], rows[0], sem_g[0])
            pltpu.async_copy(tab_c.at[src_v.at[1]], rows[1], sem_g[1])

            def ring(i0, carry):
                for b in range(4):
                    i = i0 + b
                    bn = (b + 2) % 4

                    @pl.when(i >= 2)
                    def _():
                        pltpu.make_async_copy(
                            rows[bn], acc_sh.at[dst_v.at[i - 2]],
                            sem_s[bn]).wait()

                    @pl.when(i + 2 < STEPS)
                    def _():
                        pltpu.async_copy(
                            tab_c.at[src_v.at[i + 2]], rows[bn], sem_g[bn])

                    pltpu.make_async_copy(
                        tab_c.at[src_v.at[i]], rows[b], sem_g[b]).wait()
                    pltpu.async_copy(
                        rows[b], acc_sh.at[dst_v.at[i]], sem_s[b],
                        add=True)
                return carry

            lax.fori_loop(0, STEPS // 4, lambda j, cr: ring(j * 4, cr), 0,
                          unroll=False)
            pltpu.make_async_copy(
                rows[2], acc_sh.at[dst_v.at[STEPS - 2]], sem_s[2]).wait()
            pltpu.make_async_copy(
                rows[3], acc_sh.at[dst_v.at[STEPS - 1]], sem_s[3]).wait()
            plsc.subcore_barrier()

            def cout(j, carry):
                sl = pl.ds(s * RPT + j * ZROWS, ZROWS)
                pltpu.sync_copy(acc_sh.at[sl], out_hbm.at[chunk].at[sl])
                return carry

            lax.fori_loop(0, RPT // ZROWS, cout, 0)
            if ki + 1 < K:
                plsc.subcore_barrier()

    return k(table, src2, dst2)


def _tc_colmax(x, t):
    D = x.shape[1]

    def body(x_ref, t_ref, m_ref, mx_ref):
        i = pl.program_id(0)
        pm = jnp.max(x_ref[...] * t_ref[...], axis=0, keepdims=True)

        @pl.when(i == 0)
        def _():
            mx_ref[...] = pm

        @pl.when(i > 0)
        def _():
            mx_ref[...] = jnp.maximum(mx_ref[...], pm)

        m_ref[...] = mx_ref[...]

    return pl.pallas_call(
        body,
        grid=(NB,),
        in_specs=[
            pl.BlockSpec((BR, D), lambda i: (i, 0)),
            pl.BlockSpec((1, D), lambda i: (0, 0)),
        ],
        out_specs=pl.BlockSpec((1, D), lambda i: (0, 0)),
        out_shape=jax.ShapeDtypeStruct((1, D), jnp.float32),
        scratch_shapes=[pltpu.VMEM((1, D), jnp.float32)],
    )(x, t)


def _tc_table(h, t, M, K):
    KH = K * CC // 128

    def body(h_ref, t_ref, m_ref, o_ref):
        jj = pl.program_id(1)
        hb = h_ref[...]
        e = jnp.exp(hb * t_ref[...] - m_ref[...])
        val = jnp.where(jj < KH, e, hb * e)
        o_ref[0] = val[:, :CC]
        o_ref[1] = val[:, CC:]

    return pl.pallas_call(
        body,
        grid=(NB, 2 * KH),
        in_specs=[
            pl.BlockSpec((BR, 128), lambda i, jj: (i, lax.rem(jj, KH))),
            pl.BlockSpec((1, 128), lambda i, jj: (0, lax.rem(jj, KH))),
            pl.BlockSpec((1, 128), lambda i, jj: (0, lax.rem(jj, KH))),
        ],
        out_specs=pl.BlockSpec((2, BR, CC), lambda i, jj: (jj, i, 0)),
        out_shape=jax.ShapeDtypeStruct((2 * K, N, CC), jnp.float32),
    )(h, t, M)


def _tc_sage(SA, x, Wl, b, Wr, K, H):
    D = K * CC

    def body(sa_ref, x_ref, wl_ref, b_ref, wr_ref, z_ref, s_ref, q_ref,
             acc_ref):
        i = pl.program_id(0)
        z = jnp.dot(x_ref[...], wr_ref[...],
                    preferred_element_type=jnp.float32)
        for ki in range(K):
            aggr = sa_ref[K + ki] / (sa_ref[ki] + 1e-16)
            z += jnp.dot(aggr, wl_ref[pl.ds(ki * CC, CC), :],
                         preferred_element_type=jnp.float32)
        z += b_ref[...]
        z_ref[...] = z
        ps = jnp.sum(z)
        pq = jnp.sum(z * z)

        @pl.when(i == 0)
        def _():
            acc_ref[0] = ps
            acc_ref[1] = pq

        @pl.when(i > 0)
        def _():
            acc_ref[0] += ps
            acc_ref[1] += pq

        s_ref[0, 0] = acc_ref[0]
        q_ref[0, 0] = acc_ref[1]

    return pl.pallas_call(
        body,
        grid=(NB,),
        in_specs=[
            pl.BlockSpec((2 * K, BR, CC), lambda i: (0, i, 0)),
            pl.BlockSpec((BR, D), lambda i: (i, 0)),
            pl.BlockSpec((D, H), lambda i: (0, 0)),
            pl.BlockSpec((1, H), lambda i: (0, 0)),
            pl.BlockSpec((D, H), lambda i: (0, 0)),
        ],
        out_specs=[
            pl.BlockSpec((BR, H), lambda i: (i, 0)),
            pl.BlockSpec(memory_space=pltpu.SMEM),
            pl.BlockSpec(memory_space=pltpu.SMEM),
        ],
        out_shape=[
            jax.ShapeDtypeStruct((N, H), jnp.float32),
            jax.ShapeDtypeStruct((1, 1), jnp.float32),
            jax.ShapeDtypeStruct((1, 1), jnp.float32),
        ],
        scratch_shapes=[pltpu.SMEM((2,), jnp.float32)],
    )(SA, x, Wl, b, Wr)


def _graph_ln(z_ref, s_ref, q_ref, w_ref, bb_ref, cnt):
    mean = s_ref[0, 0] / cnt
    var = q_ref[0, 0] / cnt - mean * mean
    std = jnp.sqrt(jnp.maximum(var, 0.0))
    zc = (z_ref[...] - mean) / (std + 1e-5)
    return jnp.maximum(zc * w_ref[...] + bb_ref[...], 0.0)


def _tc_ln_relu_colmax(Z, s, q, ln_w, ln_b, t, H):
    cnt = float(N * H)

    def body(z_ref, s_ref, q_ref, w_ref, bb_ref, t_ref, h_ref, m_ref, mx_ref):
        i = pl.program_id(0)
        h = _graph_ln(z_ref, s_ref, q_ref, w_ref, bb_ref, cnt)
        h_ref[...] = h
        pm = jnp.max(h * t_ref[...], axis=0, keepdims=True)

        @pl.when(i == 0)
        def _():
            mx_ref[...] = pm

        @pl.when(i > 0)
        def _():
            mx_ref[...] = jnp.maximum(mx_ref[...], pm)

        m_ref[...] = mx_ref[...]

    return pl.pallas_call(
        body,
        grid=(NB,),
        in_specs=[
            pl.BlockSpec((BR, H), lambda i: (i, 0)),
            pl.BlockSpec(memory_space=pltpu.SMEM),
            pl.BlockSpec(memory_space=pltpu.SMEM),
            pl.BlockSpec((1, H), lambda i: (0, 0)),
            pl.BlockSpec((1, H), lambda i: (0, 0)),
            pl.BlockSpec((1, H), lambda i: (0, 0)),
        ],
        out_specs=[
            pl.BlockSpec((BR, H), lambda i: (i, 0)),
            pl.BlockSpec((1, H), lambda i: (0, 0)),
        ],
        out_shape=[
            jax.ShapeDtypeStruct((N, H), jnp.float32),
            jax.ShapeDtypeStruct((1, H), jnp.float32),
        ],
        scratch_shapes=[pltpu.VMEM((1, H), jnp.float32)],
    )(Z, s, q, ln_w, ln_b, t)


def _tc_ln_relu_colsum(Z, s, q, ln_w, ln_b, H):
    cnt = float(N * H)

    def body(z_ref, s_ref, q_ref, w_ref, bb_ref, cs_ref, acc_ref):
        i = pl.program_id(0)
        h = _graph_ln(z_ref, s_ref, q_ref, w_ref, bb_ref, cnt)
        pc = jnp.sum(h, axis=0, keepdims=True)

        @pl.when(i == 0)
        def _():
            acc_ref[...] = pc

        @pl.when(i > 0)
        def _():
            acc_ref[...] += pc

        cs_ref[...] = acc_ref[...]

    return pl.pallas_call(
        body,
        grid=(NB,),
        in_specs=[
            pl.BlockSpec((BR, H), lambda i: (i, 0)),
            pl.BlockSpec(memory_space=pltpu.SMEM),
            pl.BlockSpec(memory_space=pltpu.SMEM),
            pl.BlockSpec((1, H), lambda i: (0, 0)),
            pl.BlockSpec((1, H), lambda i: (0, 0)),
        ],
        out_specs=pl.BlockSpec((1, H), lambda i: (0, 0)),
        out_shape=jax.ShapeDtypeStruct((1, H), jnp.float32),
        scratch_shapes=[pltpu.VMEM((1, H), jnp.float32)],
    )(Z, s, q, ln_w, ln_b)


def _tc_head(h2sum, fx_w, fx_b, nx_w, nx_b):
    OUT = fx_w.shape[1]

    def body(cs_ref, w_ref, b_ref, nw_ref, nb_ref, y_ref):
        g = cs_ref[...] / float(N)
        y = jnp.sum(w_ref[...] * g[0][:, None], axis=0,
                    keepdims=True) + b_ref[...]
        mu = jnp.mean(y)
        var = jnp.mean((y - mu) * (y - mu))
        y = (y - mu) / jnp.sqrt(var + 1e-5) * nw_ref[...] + nb_ref[...]
        y_ref[...] = jnp.maximum(y, 0.0)

    return pl.pallas_call(
        body,
        out_shape=jax.ShapeDtypeStruct((1, OUT), jnp.float32),
    )(h2sum, fx_w, fx_b, nx_w, nx_b)


def kernel(x, edge_index, t1, W1l, b1, W1r, ln1_w, ln1_b, t2, W2l, b2, W2r,
           ln2_w, ln2_b, fx_w, fx_b, nx_w, nx_b):
    src2 = edge_index[0].reshape(NTEC, STEPS, BATCH)
    dst2 = edge_index[1].reshape(NTEC, STEPS, BATCH)
    r2 = lambda v: v.reshape(1, -1)

    M1 = _tc_colmax(x, t1)
    T1 = _tc_table(x, t1, M1, K=2)
    SA1 = _sc_segsum(T1, src2, dst2, K=2)
    Z1, s1, q1 = _tc_sage(SA1, x, W1l, r2(b1), W1r, K=2, H=512)
    h1, M2 = _tc_ln_relu_colmax(Z1, s1, q1, r2(ln1_w), r2(ln1_b), t2, H=512)

    T2 = _tc_table(h1, t2, M2, K=8)
    SA2 = _sc_segsum(T2, src2, dst2, K=8)
    Z2, s2, q2 = _tc_sage(SA2, h1, W2l, r2(b2), W2r, K=8, H=256)
    h2sum = _tc_ln_relu_colsum(Z2, s2, q2, r2(ln2_w), r2(ln2_b), H=256)

    return _tc_head(h2sum, fx_w, r2(fx_b), r2(nx_w), r2(nx_b))


# speedup vs baseline: 5.2823x; 1.0613x over previous
import functools

import jax
import jax.numpy as jnp
from jax import lax
from jax.experimental import pallas as pl
from jax.experimental.pallas import tpu as pltpu
from jax.experimental.pallas import tpu_sc as plsc

N = 10000
E = 320000
CC = 64
NSC = 2
NTEC = 16
EPT = E // NTEC
BATCH = 125
STEPS = EPT // BATCH
RPT = N // NTEC
ZROWS = 125
BR = 1000
NB = N // BR


def _sc_segsum(table, src2, dst2, K):
    mesh = plsc.VectorSubcoreMesh(core_axis_name="c", subcore_axis_name="s")

    @functools.partial(
        pl.kernel,
        out_type=jax.ShapeDtypeStruct((2 * K, N, CC), jnp.float32),
        mesh=mesh,
        scratch_types=[
            pltpu.VMEM((STEPS, BATCH), jnp.int32),
            pltpu.VMEM((STEPS, BATCH), jnp.int32),
            [pltpu.VMEM((BATCH, CC), jnp.float32) for _ in range(5)],
            pltpu.VMEM((ZROWS, CC), jnp.float32),
            pltpu.VMEM_SHARED((N, CC), jnp.float32),
            [pltpu.SemaphoreType.DMA for _ in range(5)],
            [pltpu.SemaphoreType.DMA for _ in range(5)],
        ],
        compiler_params=pltpu.CompilerParams(use_tc_tiling_on_sc=False),
    )
    def k(tab_hbm, src_hbm, dst_hbm, out_hbm, src_v, dst_v, rows, zero_v,
          acc_sh, sem_g, sem_s):
        c = lax.axis_index("c")
        s = lax.axis_index("s")
        pltpu.sync_copy(src_hbm.at[s], src_v)
        pltpu.sync_copy(dst_hbm.at[s], dst_v)

        zeros16 = jnp.zeros((16,), jnp.float32)

        def zrow(r, carry):
            def zcol(cc, carry2):
                zero_v[r, pl.ds(cc * 16, 16)] = zeros16
                return carry2
            return lax.fori_loop(0, CC // 16, zcol, carry)

        lax.fori_loop(0, ZROWS, zrow, 0)

        for ki in range(K):
            chunk = c * K + ki
            tab_c = tab_hbm.at[chunk]

            def zinit(j, carry):
                pltpu.sync_copy(
                    zero_v, acc_sh.at[pl.ds(s * RPT + j * ZROWS, ZROWS)])
                return carry

            lax.fori_loop(0, RPT // ZROWS, zinit, 0)
            plsc.subcore_barrier()

            pltpu.async_copy(tab_c.at[src_v.at[0]], rows[0], sem_g[0])
            pltpu.async_copy(tab_c.at[src_v.at[1]], rows[1], sem_g[1])
            pltpu.async_copy(tab_c.at[src_v.at[2]], rows[2], sem_g[2])

            def ring(i0, carry):
                for b in range(5):
                    i = i0 + b
                    bn = (b + 3) % 5

                    @pl.when(i >= 2)
                    def _():
                        pltpu.make_async_copy(
                            rows[bn], acc_sh.at[dst_v.at[i - 2]],
                            sem_s[bn]).wait()

                    @pl.when(i + 3 < STEPS)
                    def _():
                        pltpu.async_copy(
                            tab_c.at[src_v.at[i + 3]], rows[bn], sem_g[bn])

                    pltpu.make_async_copy(
                        tab_c.at[src_v.at[i]], rows[b], sem_g[b]).wait()
                    pltpu.async_copy(
                        rows[b], acc_sh.at[dst_v.at[i]], sem_s[b],
                        add=True)
                return carry

            lax.fori_loop(0, STEPS // 5, lambda j, cr: ring(j * 5, cr), 0,
                          unroll=False)
            pltpu.make_async_copy(
                rows[3], acc_sh.at[dst_v.at[STEPS - 2]], sem_s[3]).wait()
            pltpu.make_async_copy(
                rows[4], acc_sh.at[dst_v.at[STEPS - 1]], sem_s[4]).wait()
            plsc.subcore_barrier()

            def cout(j, carry):
                sl = pl.ds(s * RPT + j * ZROWS, ZROWS)
                pltpu.sync_copy(acc_sh.at[sl], out_hbm.at[chunk].at[sl])
                return carry

            lax.fori_loop(0, RPT // ZROWS, cout, 0)
            if ki + 1 < K:
                plsc.subcore_barrier()

    return k(table, src2, dst2)


def _tc_colmax(x, t):
    D = x.shape[1]

    def body(x_ref, t_ref, m_ref, mx_ref):
        i = pl.program_id(0)
        pm = jnp.max(x_ref[...] * t_ref[...], axis=0, keepdims=True)

        @pl.when(i == 0)
        def _():
            mx_ref[...] = pm

        @pl.when(i > 0)
        def _():
            mx_ref[...] = jnp.maximum(mx_ref[...], pm)

        m_ref[...] = mx_ref[...]

    return pl.pallas_call(
        body,
        grid=(NB,),
        in_specs=[
            pl.BlockSpec((BR, D), lambda i: (i, 0)),
            pl.BlockSpec((1, D), lambda i: (0, 0)),
        ],
        out_specs=pl.BlockSpec((1, D), lambda i: (0, 0)),
        out_shape=jax.ShapeDtypeStruct((1, D), jnp.float32),
        scratch_shapes=[pltpu.VMEM((1, D), jnp.float32)],
    )(x, t)


def _tc_table(h, t, M, K):
    KH = K * CC // 128

    def body(h_ref, t_ref, m_ref, o_ref):
        jj = pl.program_id(1)
        hb = h_ref[...]
        e = jnp.exp(hb * t_ref[...] - m_ref[...])
        val = jnp.where(jj < KH, e, hb * e)
        o_ref[0] = val[:, :CC]
        o_ref[1] = val[:, CC:]

    return pl.pallas_call(
        body,
        grid=(NB, 2 * KH),
        in_specs=[
            pl.BlockSpec((BR, 128), lambda i, jj: (i, lax.rem(jj, KH))),
            pl.BlockSpec((1, 128), lambda i, jj: (0, lax.rem(jj, KH))),
            pl.BlockSpec((1, 128), lambda i, jj: (0, lax.rem(jj, KH))),
        ],
        out_specs=pl.BlockSpec((2, BR, CC), lambda i, jj: (jj, i, 0)),
        out_shape=jax.ShapeDtypeStruct((2 * K, N, CC), jnp.float32),
    )(h, t, M)


def _tc_sage(SA, x, Wl, b, Wr, K, H):
    D = K * CC

    def body(sa_ref, x_ref, wl_ref, b_ref, wr_ref, z_ref, s_ref, q_ref,
             acc_ref):
        i = pl.program_id(0)
        z = jnp.dot(x_ref[...], wr_ref[...],
                    preferred_element_type=jnp.float32)
        for ki in range(K):
            aggr = sa_ref[K + ki] / (sa_ref[ki] + 1e-16)
            z += jnp.dot(aggr, wl_ref[pl.ds(ki * CC, CC), :],
                         preferred_element_type=jnp.float32)
        z += b_ref[...]
        z_ref[...] = z
        ps = jnp.sum(z)
        pq = jnp.sum(z * z)

        @pl.when(i == 0)
        def _():
            acc_ref[0] = ps
            acc_ref[1] = pq

        @pl.when(i > 0)
        def _():
            acc_ref[0] += ps
            acc_ref[1] += pq

        s_ref[0, 0] = acc_ref[0]
        q_ref[0, 0] = acc_ref[1]

    return pl.pallas_call(
        body,
        grid=(NB,),
        in_specs=[
            pl.BlockSpec((2 * K, BR, CC), lambda i: (0, i, 0)),
            pl.BlockSpec((BR, D), lambda i: (i, 0)),
            pl.BlockSpec((D, H), lambda i: (0, 0)),
            pl.BlockSpec((1, H), lambda i: (0, 0)),
            pl.BlockSpec((D, H), lambda i: (0, 0)),
        ],
        out_specs=[
            pl.BlockSpec((BR, H), lambda i: (i, 0)),
            pl.BlockSpec(memory_space=pltpu.SMEM),
            pl.BlockSpec(memory_space=pltpu.SMEM),
        ],
        out_shape=[
            jax.ShapeDtypeStruct((N, H), jnp.float32),
            jax.ShapeDtypeStruct((1, 1), jnp.float32),
            jax.ShapeDtypeStruct((1, 1), jnp.float32),
        ],
        scratch_shapes=[pltpu.SMEM((2,), jnp.float32)],
    )(SA, x, Wl, b, Wr)


def _graph_ln(z_ref, s_ref, q_ref, w_ref, bb_ref, cnt):
    mean = s_ref[0, 0] / cnt
    var = q_ref[0, 0] / cnt - mean * mean
    std = jnp.sqrt(jnp.maximum(var, 0.0))
    zc = (z_ref[...] - mean) / (std + 1e-5)
    return jnp.maximum(zc * w_ref[...] + bb_ref[...], 0.0)


def _tc_ln_relu_colmax(Z, s, q, ln_w, ln_b, t, H):
    cnt = float(N * H)

    def body(z_ref, s_ref, q_ref, w_ref, bb_ref, t_ref, h_ref, m_ref, mx_ref):
        i = pl.program_id(0)
        h = _graph_ln(z_ref, s_ref, q_ref, w_ref, bb_ref, cnt)
        h_ref[...] = h
        pm = jnp.max(h * t_ref[...], axis=0, keepdims=True)

        @pl.when(i == 0)
        def _():
            mx_ref[...] = pm

        @pl.when(i > 0)
        def _():
            mx_ref[...] = jnp.maximum(mx_ref[...], pm)

        m_ref[...] = mx_ref[...]

    return pl.pallas_call(
        body,
        grid=(NB,),
        in_specs=[
            pl.BlockSpec((BR, H), lambda i: (i, 0)),
            pl.BlockSpec(memory_space=pltpu.SMEM),
            pl.BlockSpec(memory_space=pltpu.SMEM),
            pl.BlockSpec((1, H), lambda i: (0, 0)),
            pl.BlockSpec((1, H), lambda i: (0, 0)),
            pl.BlockSpec((1, H), lambda i: (0, 0)),
        ],
        out_specs=[
            pl.BlockSpec((BR, H), lambda i: (i, 0)),
            pl.BlockSpec((1, H), lambda i: (0, 0)),
        ],
        out_shape=[
            jax.ShapeDtypeStruct((N, H), jnp.float32),
            jax.ShapeDtypeStruct((1, H), jnp.float32),
        ],
        scratch_shapes=[pltpu.VMEM((1, H), jnp.float32)],
    )(Z, s, q, ln_w, ln_b, t)


def _tc_ln_relu_colsum(Z, s, q, ln_w, ln_b, H):
    cnt = float(N * H)

    def body(z_ref, s_ref, q_ref, w_ref, bb_ref, cs_ref, acc_ref):
        i = pl.program_id(0)
        h = _graph_ln(z_ref, s_ref, q_ref, w_ref, bb_ref, cnt)
        pc = jnp.sum(h, axis=0, keepdims=True)

        @pl.when(i == 0)
        def _():
            acc_ref[...] = pc

        @pl.when(i > 0)
        def _():
            acc_ref[...] += pc

        cs_ref[...] = acc_ref[...]

    return pl.pallas_call(
        body,
        grid=(NB,),
        in_specs=[
            pl.BlockSpec((BR, H), lambda i: (i, 0)),
            pl.BlockSpec(memory_space=pltpu.SMEM),
            pl.BlockSpec(memory_space=pltpu.SMEM),
            pl.BlockSpec((1, H), lambda i: (0, 0)),
            pl.BlockSpec((1, H), lambda i: (0, 0)),
        ],
        out_specs=pl.BlockSpec((1, H), lambda i: (0, 0)),
        out_shape=jax.ShapeDtypeStruct((1, H), jnp.float32),
        scratch_shapes=[pltpu.VMEM((1, H), jnp.float32)],
    )(Z, s, q, ln_w, ln_b)


def _tc_head(h2sum, fx_w, fx_b, nx_w, nx_b):
    OUT = fx_w.shape[1]

    def body(cs_ref, w_ref, b_ref, nw_ref, nb_ref, y_ref):
        g = cs_ref[...] / float(N)
        y = jnp.sum(w_ref[...] * g[0][:, None], axis=0,
                    keepdims=True) + b_ref[...]
        mu = jnp.mean(y)
        var = jnp.mean((y - mu) * (y - mu))
        y = (y - mu) / jnp.sqrt(var + 1e-5) * nw_ref[...] + nb_ref[...]
        y_ref[...] = jnp.maximum(y, 0.0)

    return pl.pallas_call(
        body,
        out_shape=jax.ShapeDtypeStruct((1, OUT), jnp.float32),
    )(h2sum, fx_w, fx_b, nx_w, nx_b)


def kernel(x, edge_index, t1, W1l, b1, W1r, ln1_w, ln1_b, t2, W2l, b2, W2r,
           ln2_w, ln2_b, fx_w, fx_b, nx_w, nx_b):
    src2 = edge_index[0].reshape(NTEC, STEPS, BATCH)
    dst2 = edge_index[1].reshape(NTEC, STEPS, BATCH)
    r2 = lambda v: v.reshape(1, -1)

    M1 = _tc_colmax(x, t1)
    T1 = _tc_table(x, t1, M1, K=2)
    SA1 = _sc_segsum(T1, src2, dst2, K=2)
    Z1, s1, q1 = _tc_sage(SA1, x, W1l, r2(b1), W1r, K=2, H=512)
    h1, M2 = _tc_ln_relu_colmax(Z1, s1, q1, r2(ln1_w), r2(ln1_b), t2, H=512)

    T2 = _tc_table(h1, t2, M2, K=8)
    SA2 = _sc_segsum(T2, src2, dst2, K=8)
    Z2, s2, q2 = _tc_sage(SA2, h1, W2l, r2(b2), W2r, K=8, H=256)
    h2sum = _tc_ln_relu_colsum(Z2, s2, q2, r2(ln2_w), r2(ln2_b), H=256)

    return _tc_head(h2sum, fx_w, r2(fx_b), r2(nx_w), r2(nx_b))
